# baseline (speedup 1.0000x reference)
import functools

import jax
import jax.numpy as jnp
from jax import lax
from jax.experimental import pallas as pl
from jax.experimental.pallas import tpu as pltpu
from jax.experimental.pallas import tpu_sc as plsc

NUM_BUCKETS = 2048
HALF_DIM = 32
EMBEDDING_DIM = 64
PAD_ROW = 72

_info = plsc.get_sparse_core_info()
_NC, _NS, _L = _info.num_cores, _info.num_subcores, _info.num_lanes
_NW = _NC * _NS

_BLK = 128
_DB = EMBEDDING_DIM // 8


def _proj_body(t_ref, w_ref, b_ref, o_ref):
    o_ref[...] = (
        jnp.dot(t_ref[...], w_ref[...], preferred_element_type=jnp.float32)
        + b_ref[...]
    )


def _project_table(bucket_table, W, b):
    W2 = jnp.concatenate(
        [W, jnp.zeros((HALF_DIM, PAD_ROW - EMBEDDING_DIM), jnp.float32)], axis=1
    )
    b2 = jnp.concatenate(
        [b, jnp.zeros((PAD_ROW - EMBEDDING_DIM,), jnp.float32)]
    ).reshape(1, PAD_ROW)
    return pl.pallas_call(
        _proj_body,
        out_shape=jax.ShapeDtypeStruct((NUM_BUCKETS, PAD_ROW), jnp.float32),
    )(bucket_table, W2, b2)


def _make_gather(B, L):
    assert B == _NW * _BLK and L % 2 == 0
    mesh = plsc.VectorSubcoreMesh(core_axis_name="c", subcore_axis_name="s")

    @functools.partial(
        pl.kernel,
        out_type=jax.ShapeDtypeStruct((L * _DB * _NW, 8 * _BLK), jnp.float32),
        mesh=mesh,
        scratch_types=[
            pltpu.VMEM((1, L, _BLK), jnp.int32),
            pltpu.VMEM((_BLK, PAD_ROW), jnp.float32),
            pltpu.VMEM((_BLK, PAD_ROW), jnp.float32),
            pltpu.VMEM((_DB, 8 * _BLK), jnp.float32),
            pltpu.VMEM((_DB, 8 * _BLK), jnp.float32),
            pltpu.SemaphoreType.DMA,
            pltpu.SemaphoreType.DMA,
            pltpu.SemaphoreType.DMA,
            pltpu.SemaphoreType.DMA,
        ],
        compiler_params=pltpu.CompilerParams(
            use_tc_tiling_on_sc=False, needs_layout_passes=False
        ),
    )
    def gather_kernel(p_hbm, x_hbm, out_hbm, idx_v, g0, g1, t0, t1,
                      gs0, gs1, ws0, ws1):
        wid = lax.axis_index("s") * _NC + lax.axis_index("c")
        gbufs, tbufs = (g0, g1), (t0, t1)
        gsems, wsems = (gs0, gs1), (ws0, ws1)

        pltpu.sync_copy(x_hbm.at[pl.ds(wid, 1)], idx_v)

        def and_body(r, carry):
            for g in range(_BLK // _L):
                sl = pl.ds(g * _L, _L)
                idx_v[0, r, sl] = lax.bitwise_and(idx_v[0, r, sl], NUM_BUCKETS - 1)
            return carry

        lax.fori_loop(0, L, and_body, 0)

        ii = lax.iota(jnp.int32, _L)
        rows_g = tuple(ii + g * _L for g in range(_BLK // _L))
        zero16 = jnp.full((_L,), 0, jnp.int32)
        di_consts = tuple(zero16 + di for di in range(8))

        def start_gather(l, par):
            return pltpu.async_copy(
                p_hbm.at[idx_v.at[0, l]], gbufs[par], gsems[par]
            )

        start_gather(0, 0)
        start_gather(1, 1)

        def pair_body(i, carry):
            for par in range(2):
                l = 2 * i + par
                gbuf, tbuf = gbufs[par], tbufs[par]
                pltpu.make_async_copy(
                    p_hbm.at[idx_v.at[0, l]], gbuf, gsems[par]
                ).wait()

                @pl.when(i > 0)
                def _drain():
                    for dB in range(_DB):
                        pltpu.make_async_copy(
                            tbuf.at[pl.ds(dB, 1)],
                            out_hbm.at[pl.ds(l * _DB * _NW + dB * _NW + wid, 1)],
                            wsems[par],
                        ).wait()

                def tr_body(dB, carry2):
                    d0 = jnp.full((_L,), 0, jnp.int32) + lax.shift_left(dB, 3)
                    for di in range(8):
                        dv = d0 + di_consts[di]
                        for g in range(_BLK // _L):
                            vals = plsc.load_gather(gbuf, [rows_g[g], dv])
                            tbuf[dB, pl.ds(di * _BLK + g * _L, _L)] = vals
                    return carry2

                lax.fori_loop(0, _DB, tr_body, 0)

                for dB in range(_DB):
                    pltpu.async_copy(
                        tbuf.at[pl.ds(dB, 1)],
                        out_hbm.at[pl.ds(l * _DB * _NW + dB * _NW + wid, 1)],
                        wsems[par],
                    )

                @pl.when(l + 2 < L)
                def _next():
                    start_gather(l + 2, par)
            return carry

        lax.fori_loop(0, L // 2, pair_body, 0)

        for par in range(2):
            for dB in range(_DB):
                pltpu.make_async_copy(
                    tbufs[par].at[pl.ds(dB, 1)],
                    out_hbm.at[pl.ds(dB * _NW + wid, 1)],
                    wsems[par],
                ).wait()

    return gather_kernel


def kernel(x, bucket_table, W, b):
    B, L = x.shape
    P = _project_table(bucket_table, W, b)
    xT = x.astype(jnp.int32).T.reshape(L, _NW, _BLK).transpose(1, 0, 2)
    out2 = _make_gather(B, L)(P, xT)
    out5 = out2.reshape(L, _DB, _NW, 8, _BLK)
    return out5.transpose(2, 4, 0, 1, 3).reshape(B, L, EMBEDDING_DIM)


# speedup vs baseline: 2.2545x; 2.2545x over previous
import functools

import jax
import jax.numpy as jnp
from jax import lax
from jax.experimental import pallas as pl
from jax.experimental.pallas import tpu as pltpu
from jax.experimental.pallas import tpu_sc as plsc

NUM_BUCKETS = 2048
HALF_DIM = 32
EMBEDDING_DIM = 64
PAD_ROW = 72

_info = plsc.get_sparse_core_info()
_NC, _NS, _L = _info.num_cores, _info.num_subcores, _info.num_lanes
_NW = _NC * _NS

_BLK = 128
_DB = EMBEDDING_DIM // 8


def _proj_body(t_ref, w_ref, b_ref, o_ref):
    o_ref[...] = (
        jnp.dot(t_ref[...], w_ref[...], preferred_element_type=jnp.float32)
        + b_ref[...]
    )


def _project_table(bucket_table, W, b):
    W2 = jnp.concatenate(
        [W, jnp.zeros((HALF_DIM, PAD_ROW - EMBEDDING_DIM), jnp.float32)], axis=1
    )
    b2 = jnp.concatenate(
        [b, jnp.zeros((PAD_ROW - EMBEDDING_DIM,), jnp.float32)]
    ).reshape(1, PAD_ROW)
    return pl.pallas_call(
        _proj_body,
        out_shape=jax.ShapeDtypeStruct((NUM_BUCKETS, PAD_ROW), jnp.float32),
    )(bucket_table, W2, b2)


def _make_gather(B, L):
    assert B == _NW * _BLK and L % 2 == 0
    mesh = plsc.VectorSubcoreMesh(core_axis_name="c", subcore_axis_name="s")

    @functools.partial(
        pl.kernel,
        out_type=jax.ShapeDtypeStruct((L * _DB * _NW, 8 * _BLK), jnp.float32),
        mesh=mesh,
        scratch_types=[
            pltpu.VMEM((1, L, _BLK), jnp.int32),
            pltpu.VMEM((_BLK, PAD_ROW), jnp.float32),
            pltpu.VMEM((_BLK, PAD_ROW), jnp.float32),
            pltpu.VMEM((_DB, 8 * _BLK), jnp.float32),
            pltpu.VMEM((_DB, 8 * _BLK), jnp.float32),
            pltpu.SemaphoreType.DMA,
            pltpu.SemaphoreType.DMA,
            pltpu.SemaphoreType.DMA,
            pltpu.SemaphoreType.DMA,
        ],
        compiler_params=pltpu.CompilerParams(
            use_tc_tiling_on_sc=False, needs_layout_passes=False
        ),
    )
    def gather_kernel(p_hbm, x_hbm, out_hbm, idx_v, g0, g1, t0, t1,
                      gs0, gs1, ws0, ws1):
        wid = lax.axis_index("s") * _NC + lax.axis_index("c")
        gbufs, tbufs = (g0, g1), (t0, t1)
        gsems, wsems = (gs0, gs1), (ws0, ws1)

        pltpu.sync_copy(x_hbm.at[pl.ds(wid, 1)], idx_v)

        def and_body(r, carry):
            for g in range(_BLK // _L):
                sl = pl.ds(g * _L, _L)
                idx_v[0, r, sl] = lax.bitwise_and(idx_v[0, r, sl], NUM_BUCKETS - 1)
            return carry

        lax.fori_loop(0, L, and_body, 0)

        ii = lax.iota(jnp.int32, _L)
        rows_g = tuple(ii + g * _L for g in range(_BLK // _L))
        zero16 = jnp.full((_L,), 0, jnp.int32)
        di_consts = tuple(zero16 + di for di in range(8))

        def start_gather(l, par):
            return pltpu.async_copy(
                p_hbm.at[idx_v.at[0, l]], gbufs[par], gsems[par]
            )

        start_gather(0, 0)
        start_gather(1, 1)

        def pair_body(i, carry):
            for par in range(2):
                l = 2 * i + par
                gbuf, tbuf = gbufs[par], tbufs[par]
                pltpu.make_async_copy(
                    p_hbm.at[idx_v.at[0, l]], gbuf, gsems[par]
                ).wait()

                @pl.when(i > 0)
                def _drain():
                    for dB in range(_DB):
                        pltpu.make_async_copy(
                            tbuf.at[pl.ds(dB, 1)],
                            out_hbm.at[pl.ds(l * _DB * _NW + dB * _NW + wid, 1)],
                            wsems[par],
                        ).wait()

                @plsc.parallel_loop(0, EMBEDDING_DIM, unroll=4)
                def tr_body(d):
                    dB = lax.shift_right_logical(d, 3)
                    c0 = lax.shift_left(lax.bitwise_and(d, 7), 7)
                    dv = jnp.full((_L,), 0, jnp.int32) + d
                    for g in range(_BLK // _L):
                        vals = plsc.load_gather(gbuf, [rows_g[g], dv])
                        tbuf[dB, pl.ds(c0 + g * _L, _L)] = vals

                for dB in range(_DB):
                    pltpu.async_copy(
                        tbuf.at[pl.ds(dB, 1)],
                        out_hbm.at[pl.ds(l * _DB * _NW + dB * _NW + wid, 1)],
                        wsems[par],
                    )

                @pl.when(l + 2 < L)
                def _next():
                    start_gather(l + 2, par)
            return carry

        lax.fori_loop(0, L // 2, pair_body, 0)

        for par in range(2):
            for dB in range(_DB):
                pltpu.make_async_copy(
                    tbufs[par].at[pl.ds(dB, 1)],
                    out_hbm.at[pl.ds(dB * _NW + wid, 1)],
                    wsems[par],
                ).wait()

    return gather_kernel


def kernel(x, bucket_table, W, b):
    B, L = x.shape
    P = _project_table(bucket_table, W, b)
    xT = x.astype(jnp.int32).T.reshape(L, _NW, _BLK).transpose(1, 0, 2)
    out2 = _make_gather(B, L)(P, xT)
    out5 = out2.reshape(L, _DB, _NW, 8, _BLK)
    return out5.transpose(2, 4, 0, 1, 3).reshape(B, L, EMBEDDING_DIM)


# speedup vs baseline: 4.0635x; 1.8024x over previous
import functools

import jax
import jax.numpy as jnp
from jax import lax
from jax.experimental import pallas as pl
from jax.experimental.pallas import tpu as pltpu
from jax.experimental.pallas import tpu_sc as plsc

NUM_BUCKETS = 2048
HALF_DIM = 32
EMBEDDING_DIM = 64
PAD_ROW = 72

_info = plsc.get_sparse_core_info()
_NC, _NS, _L = _info.num_cores, _info.num_subcores, _info.num_lanes
_NW = _NC * _NS

_BLK = 128
_DB = EMBEDDING_DIM // 8


def _proj_body(t_ref, w_ref, b_ref, o_ref):
    o_ref[...] = (
        jnp.dot(t_ref[...], w_ref[...], preferred_element_type=jnp.float32)
        + b_ref[...]
    )


def _project_table(bucket_table, W, b):
    W2 = jnp.concatenate(
        [W, jnp.zeros((HALF_DIM, PAD_ROW - EMBEDDING_DIM), jnp.float32)], axis=1
    )
    b2 = jnp.concatenate(
        [b, jnp.zeros((PAD_ROW - EMBEDDING_DIM,), jnp.float32)]
    ).reshape(1, PAD_ROW)
    return pl.pallas_call(
        _proj_body,
        out_shape=jax.ShapeDtypeStruct((NUM_BUCKETS, PAD_ROW), jnp.float32),
    )(bucket_table, W2, b2)


def _make_gather(B, L):
    assert B == _NW * _BLK and L % 2 == 0
    mesh = plsc.VectorSubcoreMesh(core_axis_name="c", subcore_axis_name="s")

    @functools.partial(
        pl.kernel,
        out_type=jax.ShapeDtypeStruct((L * _DB * _NW, 8 * _BLK), jnp.float32),
        mesh=mesh,
        scratch_types=[
            pltpu.VMEM((1, L, _BLK), jnp.int32),
            pltpu.VMEM((_BLK, PAD_ROW), jnp.float32),
            pltpu.VMEM((_BLK, PAD_ROW), jnp.float32),
            pltpu.VMEM((_DB, 8 * _BLK), jnp.float32),
            pltpu.VMEM((_DB, 8 * _BLK), jnp.float32),
            pltpu.SemaphoreType.DMA,
            pltpu.SemaphoreType.DMA,
            pltpu.SemaphoreType.DMA,
            pltpu.SemaphoreType.DMA,
            pltpu.VMEM_SHARED((NUM_BUCKETS, PAD_ROW), jnp.float32),
        ],
        compiler_params=pltpu.CompilerParams(
            use_tc_tiling_on_sc=False, needs_layout_passes=False
        ),
    )
    def gather_kernel(p_hbm, x_hbm, out_hbm, idx_v, g0, g1, t0, t1,
                      gs0, gs1, ws0, ws1, p_sh):
        wid = lax.axis_index("s") * _NC + lax.axis_index("c")

        @pl.when(lax.axis_index("s") == 0)
        def _stage_table():
            pltpu.sync_copy(p_hbm, p_sh)

        plsc.subcore_barrier()
        gbufs, tbufs = (g0, g1), (t0, t1)
        gsems, wsems = (gs0, gs1), (ws0, ws1)

        pltpu.sync_copy(x_hbm.at[pl.ds(wid, 1)], idx_v)

        def and_body(r, carry):
            for g in range(_BLK // _L):
                sl = pl.ds(g * _L, _L)
                idx_v[0, r, sl] = lax.bitwise_and(idx_v[0, r, sl], NUM_BUCKETS - 1)
            return carry

        lax.fori_loop(0, L, and_body, 0)

        ii = lax.iota(jnp.int32, _L)
        rows_g = tuple(ii + g * _L for g in range(_BLK // _L))
        zero16 = jnp.full((_L,), 0, jnp.int32)
        di_consts = tuple(zero16 + di for di in range(8))

        def start_gather(l, par):
            return pltpu.async_copy(
                p_sh.at[idx_v.at[0, l]], gbufs[par], gsems[par]
            )

        start_gather(0, 0)
        start_gather(1, 1)

        def pair_body(i, carry):
            for par in range(2):
                l = 2 * i + par
                gbuf, tbuf = gbufs[par], tbufs[par]
                pltpu.make_async_copy(
                    p_sh.at[idx_v.at[0, l]], gbuf, gsems[par]
                ).wait()

                @pl.when(i > 0)
                def _drain():
                    for dB in range(_DB):
                        pltpu.make_async_copy(
                            tbuf.at[pl.ds(dB, 1)],
                            out_hbm.at[pl.ds(l * _DB * _NW + dB * _NW + wid, 1)],
                            wsems[par],
                        ).wait()

                @plsc.parallel_loop(0, EMBEDDING_DIM, unroll=4)
                def tr_body(d):
                    dB = lax.shift_right_logical(d, 3)
                    c0 = lax.shift_left(lax.bitwise_and(d, 7), 7)
                    dv = jnp.full((_L,), 0, jnp.int32) + d
                    for g in range(_BLK // _L):
                        vals = plsc.load_gather(gbuf, [rows_g[g], dv])
                        tbuf[dB, pl.ds(c0 + g * _L, _L)] = vals

                for dB in range(_DB):
                    pltpu.async_copy(
                        tbuf.at[pl.ds(dB, 1)],
                        out_hbm.at[pl.ds(l * _DB * _NW + dB * _NW + wid, 1)],
                        wsems[par],
                    )

                @pl.when(l + 2 < L)
                def _next():
                    start_gather(l + 2, par)
            return carry

        lax.fori_loop(0, L // 2, pair_body, 0)

        for par in range(2):
            for dB in range(_DB):
                pltpu.make_async_copy(
                    tbufs[par].at[pl.ds(dB, 1)],
                    out_hbm.at[pl.ds(dB * _NW + wid, 1)],
                    wsems[par],
                ).wait()

    return gather_kernel


def kernel(x, bucket_table, W, b):
    B, L = x.shape
    P = _project_table(bucket_table, W, b)
    xT = x.astype(jnp.int32).T.reshape(L, _NW, _BLK).transpose(1, 0, 2)
    out2 = _make_gather(B, L)(P, xT)
    out5 = out2.reshape(L, _DB, _NW, 8, _BLK)
    return out5.transpose(2, 4, 0, 1, 3).reshape(B, L, EMBEDDING_DIM)
